# baseline (speedup 1.0000x reference)
import functools

import jax
import jax.numpy as jnp
from jax import lax
from jax.experimental import pallas as pl
from jax.experimental.pallas import tpu as pltpu
from jax.experimental.pallas import tpu_sc as plsc

N_DAYS = 2048
N_BUCKETS = 128
BATCH = 16384
MVOC_LO = 0.0
MVOC_HI = 1.0
MVOC_MEAN = 0.45

_NUM_CORES = 2
_NUM_SUBCORES = 16
_L = 16

_NBIG = 608
_NSML = 416
_EARLY_C = 0


def _do_slice(base, n, mvoc_hbm, day_hbm, bkt_hbm, shift_hbm, tilt_hbm,
              out_hbm, iv, fv, sems):
    idx_v = iv.at[pl.ds(0, n)]
    bkt_v = iv.at[pl.ds(_NBIG, n)]
    mv_v = fv.at[pl.ds(0, n)]
    s_v = fv.at[pl.ds(_NBIG, n)]
    t_v = fv.at[pl.ds(2 * _NBIG, n)]
    out_v = fv.at[pl.ds(3 * _NBIG, n)]
    sem_g = sems.at[0]
    sem_in = sems.at[1]
    sem_m = sems.at[2]

    cp_d = pltpu.async_copy(day_hbm.at[pl.ds(base, n)], idx_v, sem_in)
    cp_b = pltpu.async_copy(bkt_hbm.at[pl.ds(base, n)], bkt_v, sem_in)
    cp_m = pltpu.async_copy(mvoc_hbm.at[pl.ds(base, n)], mv_v, sem_m)
    cp_d.wait()
    cp_b.wait()

    def _idx_step(i, _):
        off = i * _L
        idx_v[pl.ds(off, _L)] = (
            idx_v[pl.ds(off, _L)] * N_BUCKETS + bkt_v[pl.ds(off, _L)])
        return _

    lax.fori_loop(0, n // _L, _idx_step, 0, unroll=4)

    cp_s = pltpu.async_copy(shift_hbm.at[idx_v], s_v, sem_g)
    cp_t = pltpu.async_copy(tilt_hbm.at[idx_v], t_v, sem_g)

    span = max(MVOC_HI - MVOC_LO, 1e-12)
    z_bar = jnp.float32((MVOC_MEAN - MVOC_LO) / span)
    inv_span = jnp.float32(1.0 / span)
    lo = jnp.float32(MVOC_LO)

    cp_m.wait()
    cp_s.wait()
    cp_t.wait()

    def _out_step(i, _):
        off = i * _L
        z = jnp.clip((mv_v[pl.ds(off, _L)] - lo) * inv_span, 0.0, 1.0)
        out_v[pl.ds(off, _L)] = (
            s_v[pl.ds(off, _L)] + t_v[pl.ds(off, _L)] * (z_bar - z))
        return _

    lax.fori_loop(0, n // _L, _out_step, 0, unroll=4)

    pltpu.sync_copy(out_v, out_hbm.at[pl.ds(base, n)])


def _sc_body(mvoc_hbm, day_hbm, bkt_hbm, shift_hbm, tilt_hbm, out_hbm,
             iv, fv, sems):
    c = lax.axis_index("c")
    s = lax.axis_index("s")
    args = (mvoc_hbm, day_hbm, bkt_hbm, shift_hbm, tilt_hbm, out_hbm,
            iv, fv, sems)

    @pl.when(c == _EARLY_C)
    def _():
        _do_slice(s * _NBIG, _NBIG, *args)

    @pl.when(c != _EARLY_C)
    def _():
        _do_slice(_NUM_SUBCORES * _NBIG + s * _NSML, _NSML, *args)


@functools.partial(jax.jit, static_argnames=())
def _run(mvoc, day_idx, bucket_idx, shift_flat, tilt_flat):
    mesh = plsc.VectorSubcoreMesh(core_axis_name="c", subcore_axis_name="s")
    return pl.kernel(
        _sc_body,
        out_type=jax.ShapeDtypeStruct((BATCH,), jnp.float32),
        mesh=mesh,
        scratch_types=[
            pltpu.VMEM((2 * _NBIG,), jnp.int32),
            pltpu.VMEM((4 * _NBIG,), jnp.float32),
            pltpu.SemaphoreType.DMA((3,)),
        ],
    )(mvoc, day_idx, bucket_idx, shift_flat, tilt_flat)


def kernel(mvoc, day_idx, bucket_idx, shift, tilt):
    out = _run(
        mvoc.reshape(-1),
        day_idx.reshape(-1),
        bucket_idx.reshape(-1),
        shift.reshape(-1),
        tilt.reshape(-1),
    )
    return out.reshape(-1, 1)


# speedup vs baseline: 1.0606x; 1.0606x over previous
import functools

import jax
import jax.numpy as jnp
from jax import lax
from jax.experimental import pallas as pl
from jax.experimental.pallas import tpu as pltpu
from jax.experimental.pallas import tpu_sc as plsc

N_DAYS = 2048
N_BUCKETS = 128
BATCH = 16384
MVOC_LO = 0.0
MVOC_HI = 1.0
MVOC_MEAN = 0.45

_NUM_CORES = 2
_NUM_SUBCORES = 16
_NW = _NUM_CORES * _NUM_SUBCORES
_BPW = BATCH // _NW
_L = 16


def _sc_body(mvoc_hbm, day_hbm, bkt_hbm, shift_hbm, tilt_hbm, out_hbm,
             iv, fv, sems):
    wid = lax.axis_index("s") * _NUM_CORES + lax.axis_index("c")
    base = wid * _BPW

    idx_v = iv.at[pl.ds(0, _BPW)]
    bkt_v = iv.at[pl.ds(_BPW, _BPW)]
    mv_v = fv.at[pl.ds(0, _BPW)]
    s_v = fv.at[pl.ds(_BPW, _BPW)]
    t_v = fv.at[pl.ds(2 * _BPW, _BPW)]
    out_v = fv.at[pl.ds(3 * _BPW, _BPW)]

    cp_d = pltpu.async_copy(day_hbm.at[pl.ds(base, _BPW)], idx_v, sems.at[2])
    cp_b = pltpu.async_copy(bkt_hbm.at[pl.ds(base, _BPW)], bkt_v, sems.at[2])
    cp_m = pltpu.async_copy(mvoc_hbm.at[pl.ds(base, _BPW)], mv_v, sems.at[3])
    cp_d.wait()
    cp_b.wait()

    def _idx_step(i, _):
        off = i * _L
        idx_v[pl.ds(off, _L)] = (
            idx_v[pl.ds(off, _L)] * N_BUCKETS + bkt_v[pl.ds(off, _L)])
        return _

    lax.fori_loop(0, _BPW // _L, _idx_step, 0, unroll=4)

    cp_s = pltpu.async_copy(shift_hbm.at[idx_v], s_v, sems.at[0])
    cp_t = pltpu.async_copy(tilt_hbm.at[idx_v], t_v, sems.at[1])

    span = max(MVOC_HI - MVOC_LO, 1e-12)
    z_bar = jnp.float32((MVOC_MEAN - MVOC_LO) / span)
    inv_span = jnp.float32(1.0 / span)
    lo = jnp.float32(MVOC_LO)

    cp_m.wait()
    cp_s.wait()
    cp_t.wait()

    def _out_step(i, _):
        off = i * _L
        z = jnp.clip((mv_v[pl.ds(off, _L)] - lo) * inv_span, 0.0, 1.0)
        out_v[pl.ds(off, _L)] = (
            s_v[pl.ds(off, _L)] + t_v[pl.ds(off, _L)] * (z_bar - z))
        return _

    lax.fori_loop(0, _BPW // _L, _out_step, 0, unroll=4)

    pltpu.sync_copy(out_v, out_hbm.at[pl.ds(base, _BPW)])


@functools.partial(jax.jit, static_argnames=())
def _run(mvoc, day_idx, bucket_idx, shift_flat, tilt_flat):
    mesh = plsc.VectorSubcoreMesh(core_axis_name="c", subcore_axis_name="s")
    return pl.kernel(
        _sc_body,
        out_type=jax.ShapeDtypeStruct((BATCH,), jnp.float32),
        mesh=mesh,
        scratch_types=[
            pltpu.VMEM((2 * _BPW,), jnp.int32),
            pltpu.VMEM((4 * _BPW,), jnp.float32),
            pltpu.SemaphoreType.DMA((4,)),
        ],
    )(mvoc, day_idx, bucket_idx, shift_flat, tilt_flat)


def kernel(mvoc, day_idx, bucket_idx, shift, tilt):
    out = _run(
        mvoc.reshape(-1),
        day_idx.reshape(-1),
        bucket_idx.reshape(-1),
        shift.reshape(-1),
        tilt.reshape(-1),
    )
    return out.reshape(-1, 1)
